# baseline (speedup 1.0000x reference)
import functools

import jax
import jax.numpy as jnp
from jax import lax
from jax.experimental import pallas as pl
from jax.experimental.pallas import tpu as pltpu
from jax.experimental.pallas import tpu_sc as plsc

BF = jnp.bfloat16
F32 = jnp.float32

K_CODES = 1024
D_CODE = 256
N_IMG = 16
N_TOK = 3136
VQ_BLK = 392
B_PAD = 3328


def _quadify(y):
    ho, wo, c = y.shape
    hq, wq = ho // 2 + 2, wo // 2 + 1
    yp = jnp.pad(y, ((1, 3), (1, 1), (0, 0)))
    q = yp.reshape(hq, 2, wq, 2, c).transpose(1, 3, 0, 2, 4)
    return q.reshape(4, hq * wq, c)


GB = 4


def _mm_body(x_ref, w_ref, b_ref, o_ref, *, act, out_tr, quad_hw):
    for g in range(GB):
        acc = jnp.dot(x_ref[g], w_ref[...], preferred_element_type=F32)
        acc = acc + b_ref[...]
        if act == 'relu':
            acc = jnp.maximum(acc, 0.0)
        elif act == 'sigmoid':
            acc = jax.nn.sigmoid(acc)
        if quad_hw is not None:
            ho, wo = quad_hw
            o_ref[g] = _quadify(acc.astype(o_ref.dtype).reshape(ho, wo, -1))
        elif out_tr:
            o_ref[g] = acc.T.astype(o_ref.dtype)
        else:
            o_ref[g] = acc.astype(o_ref.dtype)


def _mm_layer(xcol, w, b, act, out_dtype, out_tr=False, quad_hw=None):
    n, p, k = xcol.shape
    _, c = w.shape
    if quad_hw is not None:
        ho, wo = quad_hw
        r = (ho // 2 + 2) * (wo // 2 + 1)
        od = (n, 4, r, c)
        out_spec = pl.BlockSpec((GB, 4, r, c), lambda i: (i, 0, 0, 0))
    else:
        od = (n, c, p) if out_tr else (n, p, c)
        out_spec = pl.BlockSpec((GB,) + od[1:], lambda i: (i, 0, 0))
    return pl.pallas_call(
        functools.partial(_mm_body, act=act, out_tr=out_tr, quad_hw=quad_hw),
        grid=(n // GB,),
        in_specs=[
            pl.BlockSpec((GB, p, k), lambda i: (i, 0, 0)),
            pl.BlockSpec((k, c), lambda i: (0, 0)),
            pl.BlockSpec((1, c), lambda i: (0, 0)),
        ],
        out_specs=out_spec,
        out_shape=jax.ShapeDtypeStruct(od, out_dtype),
    )(xcol, w, b.reshape(1, c))


def _enc_im2col(x_nhwc):
    n, h, w, c = x_nhwc.shape
    ho, wo = h // 2, w // 2
    hq, wq = ho + 1, wo + 1
    xp = jnp.pad(x_nhwc, ((0, 0), (1, 1), (1, 1), (0, 0)))
    if c == 1:
        q = xp.reshape(n, hq, 2, wq, 2).transpose(0, 2, 4, 1, 3)
        planes = [q[:, ki % 2, kj % 2,
                    ki // 2:ki // 2 + ho, kj // 2:kj // 2 + wo]
                  for ki in range(4) for kj in range(4)]
        return jnp.stack(planes, axis=-1).reshape(n, ho * wo, 16)
    q = xp.reshape(n, hq, 2, wq, 2, c).transpose(0, 2, 4, 1, 3, 5)
    planes = [q[:, ki % 2, kj % 2,
                ki // 2:ki // 2 + ho, kj // 2:kj // 2 + wo, :]
              for ki in range(4) for kj in range(4)]
    return jnp.concatenate(planes, axis=-1).reshape(n, ho * wo, 16 * c)


def _enc_w(w):
    return w.transpose(2, 3, 1, 0).reshape(-1, w.shape[0]).astype(BF)


def _dec_body(x_ref, w_ref, b_ref, o_ref, *, h, w, act, inter):
    wp = w + 2
    l = h * wp
    for g in range(GB):
        acc = None
        for di in range(3):
            for dj in range(3):
                s = di * 3 + dj
                off = di * wp + dj
                part = jnp.dot(x_ref[g, off:off + l, :], w_ref[s],
                               preferred_element_type=F32)
                acc = part if acc is None else acc + part
        acc = acc + b_ref[...]
        if act == 'relu':
            acc = jnp.maximum(acc, 0.0)
        elif act == 'sigmoid':
            acc = jax.nn.sigmoid(acc)
        if inter == 'img':
            y = acc.reshape(h, wp, 2, 2).transpose(0, 2, 1, 3)
            o_ref[g] = y.reshape(2 * h, 2 * wp)[:, :2 * w]
        elif inter:
            c = acc.shape[1] // 4
            y = acc.astype(o_ref.dtype).reshape(h, wp, 2, 2, c)
            y = y.transpose(0, 2, 1, 3, 4).reshape(2 * h, 2 * wp, c)
            o_ref[g] = y[:, :2 * w, :]
        else:
            o_ref[g] = acc.astype(o_ref.dtype)


def _dec_layer(x_nhwc, w9, b, act, out_dtype, inter=False):
    n, h, w, c = x_nhwc.shape
    wp = w + 2
    r = (h + 3) * wp
    l = h * wp
    _, _, c4 = w9.shape
    cout = c4 // 4
    xf = jnp.pad(x_nhwc, ((0, 0), (1, 2), (1, 1), (0, 0))).reshape(n, r, c)
    if inter == 'img':
        out_spec = pl.BlockSpec((GB, 2 * h, 2 * w), lambda i: (i, 0, 0))
        out_shape = jax.ShapeDtypeStruct((n, 2 * h, 2 * w), out_dtype)
    elif inter:
        out_spec = pl.BlockSpec((GB, 2 * h, 2 * w, cout),
                                lambda i: (i, 0, 0, 0))
        out_shape = jax.ShapeDtypeStruct((n, 2 * h, 2 * w, cout), out_dtype)
    else:
        out_spec = pl.BlockSpec((GB, l, c4), lambda i: (i, 0, 0))
        out_shape = jax.ShapeDtypeStruct((n, l, c4), out_dtype)
    return pl.pallas_call(
        functools.partial(_dec_body, h=h, w=w, act=act, inter=inter),
        grid=(n // GB,),
        in_specs=[
            pl.BlockSpec((GB, r, c), lambda i: (i, 0, 0)),
            pl.BlockSpec((9, c, c4), lambda i: (0, 0, 0)),
            pl.BlockSpec((1, c4), lambda i: (0, 0)),
        ],
        out_specs=out_spec,
        out_shape=out_shape,
    )(xf, w9, b.reshape(1, c4))


def _dec_post(y, h, w, cout):
    n = y.shape[0]
    y = y.reshape(n, h, w + 2, 2, 2, cout).transpose(0, 1, 3, 2, 4, 5)
    return y.reshape(n, 2 * h, 2 * (w + 2), cout)[:, :, :2 * w, :]


def _enc_quad(x_nhwc):
    n, h, w, c = x_nhwc.shape
    ho, wo = h // 2, w // 2
    hq, wq = ho + 2, wo + 1
    xp = jnp.pad(x_nhwc, ((0, 0), (1, 3), (1, 1), (0, 0)))
    q = xp.reshape(n, hq, 2, wq, 2, c).transpose(0, 2, 4, 1, 3, 5)
    return q.reshape(n, 4, hq * wq, c)


def _enc_w4(w):
    cout, cin = w.shape[0], w.shape[1]
    w6 = w.transpose(2, 3, 1, 0).reshape(2, 2, 2, 2, cin, cout)
    return w6.transpose(1, 3, 0, 2, 4, 5).reshape(4, 4, cin, cout).astype(BF)


def _enc_body(x_ref, w_ref, b_ref, o_ref, *, ho, wo, act, out_tr, quad_out):
    wq = wo + 1
    l = ho * wq
    for g in range(GB):
        acc = None
        for ki in range(4):
            for kj in range(4):
                qi = (ki % 2) * 2 + (kj % 2)
                ti = (ki // 2) * 2 + (kj // 2)
                off = (ki // 2) * wq + (kj // 2)
                part = jnp.dot(x_ref[g, qi, off:off + l, :], w_ref[qi, ti],
                               preferred_element_type=F32)
                acc = part if acc is None else acc + part
        acc = acc + b_ref[...]
        if act == 'relu':
            acc = jnp.maximum(acc, 0.0)
        c = acc.shape[1]
        y = acc.astype(o_ref.dtype).reshape(ho, wq, c)[:, :wo, :]
        if out_tr:
            o_ref[g] = y.reshape(ho * wo, c).T
        elif quad_out:
            o_ref[g] = _quadify(y)
        else:
            o_ref[g] = y


def _enc_layer(xq, ho, wo, w4a, b, act, out_dtype, out_tr=False,
               quad_out=False):
    n, _, r, c = xq.shape
    cout = w4a.shape[-1]
    if out_tr:
        out_spec = pl.BlockSpec((GB, cout, ho * wo), lambda i: (i, 0, 0))
        out_shape = jax.ShapeDtypeStruct((n, cout, ho * wo), out_dtype)
    elif quad_out:
        r2 = (ho // 2 + 2) * (wo // 2 + 1)
        out_spec = pl.BlockSpec((GB, 4, r2, cout), lambda i: (i, 0, 0, 0))
        out_shape = jax.ShapeDtypeStruct((n, 4, r2, cout), out_dtype)
    else:
        out_spec = pl.BlockSpec((GB, ho, wo, cout), lambda i: (i, 0, 0, 0))
        out_shape = jax.ShapeDtypeStruct((n, ho, wo, cout), out_dtype)
    return pl.pallas_call(
        functools.partial(_enc_body, ho=ho, wo=wo, act=act, out_tr=out_tr,
                          quad_out=quad_out),
        grid=(n // GB,),
        in_specs=[
            pl.BlockSpec((GB, 4, r, c), lambda i: (i, 0, 0, 0)),
            pl.BlockSpec((4, 4, c, cout), lambda i: (0, 0, 0, 0)),
            pl.BlockSpec((1, cout), lambda i: (0, 0)),
        ],
        out_specs=out_spec,
        out_shape=out_shape,
    )(xq, w4a, b.reshape(1, cout))


def _dec_w(dw):
    cin, cout = dw.shape[0], dw.shape[1]
    wt = jnp.flip(dw, (2, 3)).transpose(1, 0, 2, 3)
    wp = wt.transpose(2, 3, 1, 0)
    w9 = jnp.zeros((9, cin, 2, 2, cout), dw.dtype)
    for pr in (0, 1):
        for di in ((0, 1) if pr == 0 else (1, 2)):
            kh = 2 * di - pr
            for pc in (0, 1):
                for dj in ((0, 1) if pc == 0 else (1, 2)):
                    kw = 2 * dj - pc
                    w9 = w9.at[di * 3 + dj, :, pr, pc, :].set(wp[kh, kw])
    return w9.reshape(9, cin, 4 * cout).astype(BF)


def _vq_body(f32_ref, fbf_ref, embt_ref, emb_ref, idx_ref):
    f = f32_ref[0]
    m = jnp.dot(fbf_ref[0], embt_ref[...], preferred_element_type=F32)
    f2 = jnp.sum(f * f, axis=1, keepdims=True)
    e = emb_ref[...]
    e2 = jnp.sum(e * e, axis=1)[None, :]
    d = f2 + e2 - 2.0 * m
    dmin = jnp.min(d, axis=1, keepdims=True)
    col = lax.broadcasted_iota(jnp.int32, d.shape, 1)
    idx = jnp.min(jnp.where(d == dmin, col, K_CODES), axis=1)
    idx_ref[0] = idx.reshape(1, VQ_BLK)


def _vq_argmin():
    return pl.pallas_call(
        _vq_body,
        grid=(8,),
        in_specs=[
            pl.BlockSpec((1, VQ_BLK, D_CODE), lambda i: (i, 0, 0)),
            pl.BlockSpec((1, VQ_BLK, D_CODE), lambda i: (i, 0, 0)),
            pl.BlockSpec((D_CODE, K_CODES), lambda i: (0, 0)),
            pl.BlockSpec((K_CODES, D_CODE), lambda i: (0, 0)),
        ],
        out_specs=pl.BlockSpec((1, 1, VQ_BLK), lambda i: (i, 0, 0)),
        out_shape=jax.ShapeDtypeStruct((8, 1, VQ_BLK), jnp.int32),
    )


def _sc_gather_body(table_hbm, idx_hbm, out_hbm, idx_v, rows_v, sem, *, nc, bpw):
    wid = lax.axis_index("s") * nc + lax.axis_index("c")
    base = wid * bpw
    pltpu.sync_copy(idx_hbm.at[pl.ds(base, bpw)], idx_v)
    pltpu.async_copy(table_hbm.at[idx_v], rows_v, sem).wait()
    pltpu.sync_copy(rows_v, out_hbm.at[pl.ds(base, bpw)])


def _sc_gather(emb_f32, idx_pad):
    info = plsc.get_sparse_core_info()
    nc, ns = info.num_cores, info.num_subcores
    bpw = B_PAD // (nc * ns)
    k = pl.kernel(
        functools.partial(_sc_gather_body, nc=nc, bpw=bpw),
        mesh=plsc.VectorSubcoreMesh(core_axis_name="c", subcore_axis_name="s"),
        out_type=jax.ShapeDtypeStruct((B_PAD, D_CODE), F32),
        scratch_types=[
            pltpu.VMEM((bpw,), jnp.int32),
            pltpu.VMEM((bpw, D_CODE), F32),
            pltpu.SemaphoreType.DMA,
        ],
    )
    return k(emb_f32, idx_pad)


def _loss_body(q_ref, z_ref, part_ref, qn_ref):
    for g in range(GB):
        q = q_ref[g]
        dlt = q - z_ref[g]
        part_ref[g] = ((1.25 / (N_TOK * D_CODE))
                       * jnp.sum(dlt * dlt)).reshape(1, 1)
        qn_ref[g] = q.T.astype(BF)


def _loss_and_qt(qc, zc):
    return pl.pallas_call(
        _loss_body,
        grid=(N_IMG // GB,),
        in_specs=[
            pl.BlockSpec((GB, 256, 196), lambda i: (i, 0, 0)),
            pl.BlockSpec((GB, 256, 196), lambda i: (i, 0, 0)),
        ],
        out_specs=[
            pl.BlockSpec((GB, 1, 1), lambda i: (i, 0, 0)),
            pl.BlockSpec((GB, 196, 256), lambda i: (i, 0, 0)),
        ],
        out_shape=[
            jax.ShapeDtypeStruct((N_IMG, 1, 1), F32),
            jax.ShapeDtypeStruct((N_IMG, 196, 256), BF),
        ],
    )(qc, zc)


def kernel(x, w1, b1, w2, b2, w3, b3, w4, b4, emb,
           dw1, db1, dw2, db2, dw3, db3, dw4, db4):
    nhwc = lambda a: a.transpose(0, 2, 3, 1)

    h = _mm_layer(_enc_im2col(nhwc(x).astype(BF)), _enc_w(w1), b1, 'relu', BF,
                  quad_hw=(112, 112))
    h = _enc_layer(h, 56, 56, _enc_w4(w2), b2, 'relu', BF, quad_out=True)
    h = _enc_layer(h, 28, 28, _enc_w4(w3), b3, 'relu', BF, quad_out=True)
    zc = _enc_layer(h, 14, 14, _enc_w4(w4), b4, 'none', F32,
                    out_tr=True)

    zf = zc.reshape(N_TOK, D_CODE)
    embt = emb.T.astype(BF)
    idx8 = _vq_argmin()(zf.reshape(8, VQ_BLK, D_CODE),
                        zf.astype(BF).reshape(8, VQ_BLK, D_CODE), embt, emb)
    indices = idx8.reshape(N_TOK, 1)
    idx_pad = jnp.concatenate(
        [idx8.reshape(N_TOK), jnp.zeros((B_PAD - N_TOK,), jnp.int32)])
    qf = _sc_gather(emb, idx_pad)[:N_TOK]
    parts, q_nhwc = _loss_and_qt(qf.reshape(N_IMG, 256, 196), zc)
    vq_loss = jnp.sum(parts)

    q_nhwc = q_nhwc.reshape(N_IMG, 14, 14, 256)
    h = _dec_layer(q_nhwc, _dec_w(dw1), jnp.tile(db1, 4), 'relu', BF, inter=True)
    h = _dec_layer(h, _dec_w(dw2), jnp.tile(db2, 4), 'relu', BF, inter=True)
    h = _dec_layer(h, _dec_w(dw3), jnp.tile(db3, 4), 'relu', BF, inter=True)
    y = _dec_layer(h, _dec_w(dw4), jnp.tile(db4, 4), 'sigmoid', F32,
                   inter='img')
    x_recon = y.reshape(N_IMG, 1, 224, 224)

    return (x_recon, vq_loss, indices)


# speedup vs baseline: 1.0252x; 1.0252x over previous
import functools

import jax
import jax.numpy as jnp
from jax import lax
from jax.experimental import pallas as pl
from jax.experimental.pallas import tpu as pltpu
from jax.experimental.pallas import tpu_sc as plsc

BF = jnp.bfloat16
F32 = jnp.float32

K_CODES = 1024
D_CODE = 256
N_IMG = 16
N_TOK = 3136
VQ_BLK = 392
B_PAD = 3328


def _quadify(y):
    ho, wo, c = y.shape
    hq, wq = ho // 2 + 2, wo // 2 + 1
    yp = jnp.pad(y, ((1, 3), (1, 1), (0, 0)))
    q = yp.reshape(hq, 2, wq, 2, c).transpose(1, 3, 0, 2, 4)
    return q.reshape(4, hq * wq, c)


GB = 1


def _mm_body(x_ref, w_ref, b_ref, o_ref, *, act, out_tr, quad_hw):
    for g in range(GB):
        acc = jnp.dot(x_ref[g], w_ref[...], preferred_element_type=F32)
        acc = acc + b_ref[...]
        if act == 'relu':
            acc = jnp.maximum(acc, 0.0)
        elif act == 'sigmoid':
            acc = jax.nn.sigmoid(acc)
        if quad_hw is not None:
            ho, wo = quad_hw
            o_ref[g] = _quadify(acc.astype(o_ref.dtype).reshape(ho, wo, -1))
        elif out_tr:
            o_ref[g] = acc.T.astype(o_ref.dtype)
        else:
            o_ref[g] = acc.astype(o_ref.dtype)


def _mm_layer(xcol, w, b, act, out_dtype, out_tr=False, quad_hw=None):
    n, p, k = xcol.shape
    _, c = w.shape
    if quad_hw is not None:
        ho, wo = quad_hw
        r = (ho // 2 + 2) * (wo // 2 + 1)
        od = (n, 4, r, c)
        out_spec = pl.BlockSpec((GB, 4, r, c), lambda i: (i, 0, 0, 0))
    else:
        od = (n, c, p) if out_tr else (n, p, c)
        out_spec = pl.BlockSpec((GB,) + od[1:], lambda i: (i, 0, 0))
    return pl.pallas_call(
        functools.partial(_mm_body, act=act, out_tr=out_tr, quad_hw=quad_hw),
        grid=(n // GB,),
        in_specs=[
            pl.BlockSpec((GB, p, k), lambda i: (i, 0, 0)),
            pl.BlockSpec((k, c), lambda i: (0, 0)),
            pl.BlockSpec((1, c), lambda i: (0, 0)),
        ],
        out_specs=out_spec,
        out_shape=jax.ShapeDtypeStruct(od, out_dtype),
    )(xcol, w, b.reshape(1, c))


def _enc_im2col(x_nhwc):
    n, h, w, c = x_nhwc.shape
    ho, wo = h // 2, w // 2
    hq, wq = ho + 1, wo + 1
    xp = jnp.pad(x_nhwc, ((0, 0), (1, 1), (1, 1), (0, 0)))
    if c == 1:
        q = xp.reshape(n, hq, 2, wq, 2).transpose(0, 2, 4, 1, 3)
        planes = [q[:, ki % 2, kj % 2,
                    ki // 2:ki // 2 + ho, kj // 2:kj // 2 + wo]
                  for ki in range(4) for kj in range(4)]
        return jnp.stack(planes, axis=-1).reshape(n, ho * wo, 16)
    q = xp.reshape(n, hq, 2, wq, 2, c).transpose(0, 2, 4, 1, 3, 5)
    planes = [q[:, ki % 2, kj % 2,
                ki // 2:ki // 2 + ho, kj // 2:kj // 2 + wo, :]
              for ki in range(4) for kj in range(4)]
    return jnp.concatenate(planes, axis=-1).reshape(n, ho * wo, 16 * c)


def _enc_w(w):
    return w.transpose(2, 3, 1, 0).reshape(-1, w.shape[0]).astype(BF)


def _dec_body(x_ref, w_ref, b_ref, o_ref, *, h, w, act, inter):
    wp = w + 2
    l = h * wp
    for g in range(GB):
        acc = None
        for di in range(3):
            for dj in range(3):
                s = di * 3 + dj
                off = di * wp + dj
                part = jnp.dot(x_ref[g, off:off + l, :], w_ref[s],
                               preferred_element_type=F32)
                acc = part if acc is None else acc + part
        acc = acc + b_ref[...]
        if act == 'relu':
            acc = jnp.maximum(acc, 0.0)
        elif act == 'sigmoid':
            acc = jax.nn.sigmoid(acc)
        if inter == 'img':
            y = acc.reshape(h, wp, 2, 2).transpose(0, 2, 1, 3)
            o_ref[g] = y.reshape(2 * h, 2 * wp)[:, :2 * w]
        elif inter:
            c = acc.shape[1] // 4
            y = acc.astype(o_ref.dtype).reshape(h, wp, 2, 2, c)
            y = y.transpose(0, 2, 1, 3, 4).reshape(2 * h, 2 * wp, c)
            o_ref[g] = y[:, :2 * w, :]
        else:
            o_ref[g] = acc.astype(o_ref.dtype)


def _dec_layer(x_nhwc, w9, b, act, out_dtype, inter=False):
    n, h, w, c = x_nhwc.shape
    wp = w + 2
    r = (h + 3) * wp
    l = h * wp
    _, _, c4 = w9.shape
    cout = c4 // 4
    xf = jnp.pad(x_nhwc, ((0, 0), (1, 2), (1, 1), (0, 0))).reshape(n, r, c)
    if inter == 'img':
        out_spec = pl.BlockSpec((GB, 2 * h, 2 * w), lambda i: (i, 0, 0))
        out_shape = jax.ShapeDtypeStruct((n, 2 * h, 2 * w), out_dtype)
    elif inter:
        out_spec = pl.BlockSpec((GB, 2 * h, 2 * w, cout),
                                lambda i: (i, 0, 0, 0))
        out_shape = jax.ShapeDtypeStruct((n, 2 * h, 2 * w, cout), out_dtype)
    else:
        out_spec = pl.BlockSpec((GB, l, c4), lambda i: (i, 0, 0))
        out_shape = jax.ShapeDtypeStruct((n, l, c4), out_dtype)
    return pl.pallas_call(
        functools.partial(_dec_body, h=h, w=w, act=act, inter=inter),
        grid=(n // GB,),
        in_specs=[
            pl.BlockSpec((GB, r, c), lambda i: (i, 0, 0)),
            pl.BlockSpec((9, c, c4), lambda i: (0, 0, 0)),
            pl.BlockSpec((1, c4), lambda i: (0, 0)),
        ],
        out_specs=out_spec,
        out_shape=out_shape,
    )(xf, w9, b.reshape(1, c4))


def _dec_post(y, h, w, cout):
    n = y.shape[0]
    y = y.reshape(n, h, w + 2, 2, 2, cout).transpose(0, 1, 3, 2, 4, 5)
    return y.reshape(n, 2 * h, 2 * (w + 2), cout)[:, :, :2 * w, :]


def _enc_quad(x_nhwc):
    n, h, w, c = x_nhwc.shape
    ho, wo = h // 2, w // 2
    hq, wq = ho + 2, wo + 1
    xp = jnp.pad(x_nhwc, ((0, 0), (1, 3), (1, 1), (0, 0)))
    q = xp.reshape(n, hq, 2, wq, 2, c).transpose(0, 2, 4, 1, 3, 5)
    return q.reshape(n, 4, hq * wq, c)


def _enc_w4(w):
    cout, cin = w.shape[0], w.shape[1]
    w6 = w.transpose(2, 3, 1, 0).reshape(2, 2, 2, 2, cin, cout)
    return w6.transpose(1, 3, 0, 2, 4, 5).reshape(4, 4, cin, cout).astype(BF)


def _enc_body(x_ref, w_ref, b_ref, o_ref, *, ho, wo, act, out_tr, quad_out):
    wq = wo + 1
    l = ho * wq
    for g in range(GB):
        acc = None
        for ki in range(4):
            for kj in range(4):
                qi = (ki % 2) * 2 + (kj % 2)
                ti = (ki // 2) * 2 + (kj // 2)
                off = (ki // 2) * wq + (kj // 2)
                part = jnp.dot(x_ref[g, qi, off:off + l, :], w_ref[qi, ti],
                               preferred_element_type=F32)
                acc = part if acc is None else acc + part
        acc = acc + b_ref[...]
        if act == 'relu':
            acc = jnp.maximum(acc, 0.0)
        c = acc.shape[1]
        y = acc.astype(o_ref.dtype).reshape(ho, wq, c)[:, :wo, :]
        if out_tr:
            o_ref[g] = y.reshape(ho * wo, c).T
        elif quad_out:
            o_ref[g] = _quadify(y)
        else:
            o_ref[g] = y


def _enc_layer(xq, ho, wo, w4a, b, act, out_dtype, out_tr=False,
               quad_out=False):
    n, _, r, c = xq.shape
    cout = w4a.shape[-1]
    if out_tr:
        out_spec = pl.BlockSpec((GB, cout, ho * wo), lambda i: (i, 0, 0))
        out_shape = jax.ShapeDtypeStruct((n, cout, ho * wo), out_dtype)
    elif quad_out:
        r2 = (ho // 2 + 2) * (wo // 2 + 1)
        out_spec = pl.BlockSpec((GB, 4, r2, cout), lambda i: (i, 0, 0, 0))
        out_shape = jax.ShapeDtypeStruct((n, 4, r2, cout), out_dtype)
    else:
        out_spec = pl.BlockSpec((GB, ho, wo, cout), lambda i: (i, 0, 0, 0))
        out_shape = jax.ShapeDtypeStruct((n, ho, wo, cout), out_dtype)
    return pl.pallas_call(
        functools.partial(_enc_body, ho=ho, wo=wo, act=act, out_tr=out_tr,
                          quad_out=quad_out),
        grid=(n // GB,),
        in_specs=[
            pl.BlockSpec((GB, 4, r, c), lambda i: (i, 0, 0, 0)),
            pl.BlockSpec((4, 4, c, cout), lambda i: (0, 0, 0, 0)),
            pl.BlockSpec((1, cout), lambda i: (0, 0)),
        ],
        out_specs=out_spec,
        out_shape=out_shape,
    )(xq, w4a, b.reshape(1, cout))


def _dec_w(dw):
    cin, cout = dw.shape[0], dw.shape[1]
    wt = jnp.flip(dw, (2, 3)).transpose(1, 0, 2, 3)
    wp = wt.transpose(2, 3, 1, 0)
    w9 = jnp.zeros((9, cin, 2, 2, cout), dw.dtype)
    for pr in (0, 1):
        for di in ((0, 1) if pr == 0 else (1, 2)):
            kh = 2 * di - pr
            for pc in (0, 1):
                for dj in ((0, 1) if pc == 0 else (1, 2)):
                    kw = 2 * dj - pc
                    w9 = w9.at[di * 3 + dj, :, pr, pc, :].set(wp[kh, kw])
    return w9.reshape(9, cin, 4 * cout).astype(BF)


def _vq_body(f32_ref, fbf_ref, embt_ref, emb_ref, idx_ref):
    f = f32_ref[0]
    m = jnp.dot(fbf_ref[0], embt_ref[...], preferred_element_type=F32)
    f2 = jnp.sum(f * f, axis=1, keepdims=True)
    e = emb_ref[...]
    e2 = jnp.sum(e * e, axis=1)[None, :]
    d = f2 + e2 - 2.0 * m
    dmin = jnp.min(d, axis=1, keepdims=True)
    col = lax.broadcasted_iota(jnp.int32, d.shape, 1)
    idx = jnp.min(jnp.where(d == dmin, col, K_CODES), axis=1)
    idx_ref[0] = idx.reshape(1, VQ_BLK)


def _vq_argmin():
    return pl.pallas_call(
        _vq_body,
        grid=(8,),
        in_specs=[
            pl.BlockSpec((1, VQ_BLK, D_CODE), lambda i: (i, 0, 0)),
            pl.BlockSpec((1, VQ_BLK, D_CODE), lambda i: (i, 0, 0)),
            pl.BlockSpec((D_CODE, K_CODES), lambda i: (0, 0)),
            pl.BlockSpec((K_CODES, D_CODE), lambda i: (0, 0)),
        ],
        out_specs=pl.BlockSpec((1, 1, VQ_BLK), lambda i: (i, 0, 0)),
        out_shape=jax.ShapeDtypeStruct((8, 1, VQ_BLK), jnp.int32),
    )


def _sc_gather_body(table_hbm, idx_hbm, out_hbm, idx_v, rows_v, sem, *, nc, bpw):
    wid = lax.axis_index("s") * nc + lax.axis_index("c")
    base = wid * bpw
    pltpu.sync_copy(idx_hbm.at[pl.ds(base, bpw)], idx_v)
    pltpu.async_copy(table_hbm.at[idx_v], rows_v, sem).wait()
    pltpu.sync_copy(rows_v, out_hbm.at[pl.ds(base, bpw)])


def _sc_gather(emb_f32, idx_pad):
    info = plsc.get_sparse_core_info()
    nc, ns = info.num_cores, info.num_subcores
    bpw = B_PAD // (nc * ns)
    k = pl.kernel(
        functools.partial(_sc_gather_body, nc=nc, bpw=bpw),
        mesh=plsc.VectorSubcoreMesh(core_axis_name="c", subcore_axis_name="s"),
        out_type=jax.ShapeDtypeStruct((B_PAD, D_CODE), F32),
        scratch_types=[
            pltpu.VMEM((bpw,), jnp.int32),
            pltpu.VMEM((bpw, D_CODE), F32),
            pltpu.SemaphoreType.DMA,
        ],
    )
    return k(emb_f32, idx_pad)


def _loss_body(q_ref, z_ref, part_ref, qn_ref):
    for g in range(GB):
        q = q_ref[g]
        dlt = q - z_ref[g]
        part_ref[g] = ((1.25 / (N_TOK * D_CODE))
                       * jnp.sum(dlt * dlt)).reshape(1, 1)
        qn_ref[g] = q.T.astype(BF)


def _loss_and_qt(qc, zc):
    return pl.pallas_call(
        _loss_body,
        grid=(N_IMG // GB,),
        in_specs=[
            pl.BlockSpec((GB, 256, 196), lambda i: (i, 0, 0)),
            pl.BlockSpec((GB, 256, 196), lambda i: (i, 0, 0)),
        ],
        out_specs=[
            pl.BlockSpec((GB, 1, 1), lambda i: (i, 0, 0)),
            pl.BlockSpec((GB, 196, 256), lambda i: (i, 0, 0)),
        ],
        out_shape=[
            jax.ShapeDtypeStruct((N_IMG, 1, 1), F32),
            jax.ShapeDtypeStruct((N_IMG, 196, 256), BF),
        ],
    )(qc, zc)


def kernel(x, w1, b1, w2, b2, w3, b3, w4, b4, emb,
           dw1, db1, dw2, db2, dw3, db3, dw4, db4):
    nhwc = lambda a: a.transpose(0, 2, 3, 1)

    h = _mm_layer(_enc_im2col(nhwc(x).astype(BF)), _enc_w(w1), b1, 'relu', BF,
                  quad_hw=(112, 112))
    h = _enc_layer(h, 56, 56, _enc_w4(w2), b2, 'relu', BF, quad_out=True)
    h = _enc_layer(h, 28, 28, _enc_w4(w3), b3, 'relu', BF, quad_out=True)
    zc = _enc_layer(h, 14, 14, _enc_w4(w4), b4, 'none', F32,
                    out_tr=True)

    zf = zc.reshape(N_TOK, D_CODE)
    embt = emb.T.astype(BF)
    idx8 = _vq_argmin()(zf.reshape(8, VQ_BLK, D_CODE),
                        zf.astype(BF).reshape(8, VQ_BLK, D_CODE), embt, emb)
    indices = idx8.reshape(N_TOK, 1)
    idx_pad = jnp.concatenate(
        [idx8.reshape(N_TOK), jnp.zeros((B_PAD - N_TOK,), jnp.int32)])
    qf = _sc_gather(emb, idx_pad)[:N_TOK]
    parts, q_nhwc = _loss_and_qt(qf.reshape(N_IMG, 256, 196), zc)
    vq_loss = jnp.sum(parts)

    q_nhwc = q_nhwc.reshape(N_IMG, 14, 14, 256)
    h = _dec_layer(q_nhwc, _dec_w(dw1), jnp.tile(db1, 4), 'relu', BF, inter=True)
    h = _dec_layer(h, _dec_w(dw2), jnp.tile(db2, 4), 'relu', BF, inter=True)
    h = _dec_layer(h, _dec_w(dw3), jnp.tile(db3, 4), 'relu', BF, inter=True)
    y = _dec_layer(h, _dec_w(dw4), jnp.tile(db4, 4), 'sigmoid', F32,
                   inter='img')
    x_recon = y.reshape(N_IMG, 1, 224, 224)

    return (x_recon, vq_loss, indices)


# speedup vs baseline: 1.0357x; 1.0103x over previous
import functools

import jax
import jax.numpy as jnp
from jax import lax
from jax.experimental import pallas as pl
from jax.experimental.pallas import tpu as pltpu
from jax.experimental.pallas import tpu_sc as plsc

BF = jnp.bfloat16
F32 = jnp.float32

K_CODES = 1024
D_CODE = 256
N_IMG = 16
N_TOK = 3136
VQ_BLK = 392
B_PAD = 3328


def _quadify(y):
    ho, wo, c = y.shape
    hq, wq = ho // 2 + 2, wo // 2 + 1
    yp = jnp.pad(y, ((1, 3), (1, 1), (0, 0)))
    q = yp.reshape(hq, 2, wq, 2, c).transpose(1, 3, 0, 2, 4)
    return q.reshape(4, hq * wq, c)


GB = 1


def _mm_body(x_ref, w_ref, b_ref, o_ref, *, act, out_tr, quad_hw):
    for g in range(GB):
        acc = jnp.dot(x_ref[g], w_ref[...], preferred_element_type=F32)
        acc = acc + b_ref[...]
        if act == 'relu':
            acc = jnp.maximum(acc, 0.0)
        elif act == 'sigmoid':
            acc = jax.nn.sigmoid(acc)
        if quad_hw is not None:
            ho, wo = quad_hw
            o_ref[g] = _quadify(acc.astype(o_ref.dtype).reshape(ho, wo, -1))
        elif out_tr:
            o_ref[g] = acc.T.astype(o_ref.dtype)
        else:
            o_ref[g] = acc.astype(o_ref.dtype)


def _mm_layer(xcol, w, b, act, out_dtype, out_tr=False, quad_hw=None):
    n, p, k = xcol.shape
    _, c = w.shape
    if quad_hw is not None:
        ho, wo = quad_hw
        r = (ho // 2 + 2) * (wo // 2 + 1)
        od = (n, 4, r, c)
        out_spec = pl.BlockSpec((GB, 4, r, c), lambda i: (i, 0, 0, 0))
    else:
        od = (n, c, p) if out_tr else (n, p, c)
        out_spec = pl.BlockSpec((GB,) + od[1:], lambda i: (i, 0, 0))
    return pl.pallas_call(
        functools.partial(_mm_body, act=act, out_tr=out_tr, quad_hw=quad_hw),
        grid=(n // GB,),
        in_specs=[
            pl.BlockSpec((GB, p, k), lambda i: (i, 0, 0)),
            pl.BlockSpec((k, c), lambda i: (0, 0)),
            pl.BlockSpec((1, c), lambda i: (0, 0)),
        ],
        out_specs=out_spec,
        out_shape=jax.ShapeDtypeStruct(od, out_dtype),
    )(xcol, w, b.reshape(1, c))


def _enc_im2col(x_nhwc):
    n, h, w, c = x_nhwc.shape
    ho, wo = h // 2, w // 2
    hq, wq = ho + 1, wo + 1
    xp = jnp.pad(x_nhwc, ((0, 0), (1, 1), (1, 1), (0, 0)))
    if c == 1:
        q = xp.reshape(n, hq, 2, wq, 2).transpose(0, 2, 4, 1, 3)
        planes = [q[:, ki % 2, kj % 2,
                    ki // 2:ki // 2 + ho, kj // 2:kj // 2 + wo]
                  for ki in range(4) for kj in range(4)]
        return jnp.stack(planes, axis=-1).reshape(n, ho * wo, 16)
    q = xp.reshape(n, hq, 2, wq, 2, c).transpose(0, 2, 4, 1, 3, 5)
    planes = [q[:, ki % 2, kj % 2,
                ki // 2:ki // 2 + ho, kj // 2:kj // 2 + wo, :]
              for ki in range(4) for kj in range(4)]
    return jnp.concatenate(planes, axis=-1).reshape(n, ho * wo, 16 * c)


def _enc_w(w):
    return w.transpose(2, 3, 1, 0).reshape(-1, w.shape[0]).astype(BF)


def _dec_body(x_ref, w_ref, b_ref, o_ref, *, h, w, act, inter):
    wp = w + 2
    l = h * wp
    for g in range(GB):
        acc = None
        for di in range(3):
            for dj in range(3):
                s = di * 3 + dj
                off = di * wp + dj
                part = jnp.dot(x_ref[g, off:off + l, :], w_ref[s],
                               preferred_element_type=F32)
                acc = part if acc is None else acc + part
        acc = acc + b_ref[...]
        if act == 'relu':
            acc = jnp.maximum(acc, 0.0)
        elif act == 'sigmoid':
            acc = jax.nn.sigmoid(acc)
        if inter == 'img':
            y = acc.reshape(h, wp, 2, 2).transpose(0, 2, 1, 3)
            o_ref[g] = y.reshape(2 * h, 2 * wp)[:, :2 * w]
        elif inter:
            c = acc.shape[1] // 4
            y = acc.astype(o_ref.dtype).reshape(h, wp, 2, 2, c)
            y = y.transpose(0, 2, 1, 3, 4).reshape(2 * h, 2 * wp, c)
            o_ref[g] = y[:, :2 * w, :]
        else:
            o_ref[g] = acc.astype(o_ref.dtype)


def _dec_layer(x_nhwc, w9, b, act, out_dtype, inter=False):
    n, h, w, c = x_nhwc.shape
    wp = w + 2
    r = (h + 3) * wp
    l = h * wp
    _, _, c4 = w9.shape
    cout = c4 // 4
    xf = jnp.pad(x_nhwc, ((0, 0), (1, 2), (1, 1), (0, 0))).reshape(n, r, c)
    if inter == 'img':
        out_spec = pl.BlockSpec((GB, 2 * h, 2 * w), lambda i: (i, 0, 0))
        out_shape = jax.ShapeDtypeStruct((n, 2 * h, 2 * w), out_dtype)
    elif inter:
        out_spec = pl.BlockSpec((GB, 2 * h, 2 * w, cout),
                                lambda i: (i, 0, 0, 0))
        out_shape = jax.ShapeDtypeStruct((n, 2 * h, 2 * w, cout), out_dtype)
    else:
        out_spec = pl.BlockSpec((GB, l, c4), lambda i: (i, 0, 0))
        out_shape = jax.ShapeDtypeStruct((n, l, c4), out_dtype)
    return pl.pallas_call(
        functools.partial(_dec_body, h=h, w=w, act=act, inter=inter),
        grid=(n // GB,),
        in_specs=[
            pl.BlockSpec((GB, r, c), lambda i: (i, 0, 0)),
            pl.BlockSpec((9, c, c4), lambda i: (0, 0, 0)),
            pl.BlockSpec((1, c4), lambda i: (0, 0)),
        ],
        out_specs=out_spec,
        out_shape=out_shape,
    )(xf, w9, b.reshape(1, c4))


def _dec_post(y, h, w, cout):
    n = y.shape[0]
    y = y.reshape(n, h, w + 2, 2, 2, cout).transpose(0, 1, 3, 2, 4, 5)
    return y.reshape(n, 2 * h, 2 * (w + 2), cout)[:, :, :2 * w, :]


def _enc_quad(x_nhwc):
    n, h, w, c = x_nhwc.shape
    ho, wo = h // 2, w // 2
    hq, wq = ho + 2, wo + 1
    xp = jnp.pad(x_nhwc, ((0, 0), (1, 3), (1, 1), (0, 0)))
    q = xp.reshape(n, hq, 2, wq, 2, c).transpose(0, 2, 4, 1, 3, 5)
    return q.reshape(n, 4, hq * wq, c)


def _enc_w4(w):
    cout, cin = w.shape[0], w.shape[1]
    w6 = w.transpose(2, 3, 1, 0).reshape(2, 2, 2, 2, cin, cout)
    return w6.transpose(1, 3, 0, 2, 4, 5).reshape(4, 4, cin, cout).astype(BF)


def _enc_body(x_ref, w_ref, b_ref, o_ref, *, ho, wo, act, out_tr, quad_out):
    wq = wo + 1
    l = ho * wq
    for g in range(GB):
        cols = []
        for ki in range(4):
            for kj in range(4):
                qi = (ki % 2) * 2 + (kj % 2)
                off = (ki // 2) * wq + (kj // 2)
                cols.append(x_ref[g, qi, off:off + l, :])
        xcol = jnp.concatenate(cols, axis=1)
        acc = jnp.dot(xcol, w_ref[...], preferred_element_type=F32)
        acc = acc + b_ref[...]
        if act == 'relu':
            acc = jnp.maximum(acc, 0.0)
        c = acc.shape[1]
        y = acc.astype(o_ref.dtype).reshape(ho, wq, c)[:, :wo, :]
        if out_tr:
            o_ref[g] = y.reshape(ho * wo, c).T
        elif quad_out:
            o_ref[g] = _quadify(y)
        else:
            o_ref[g] = y


def _enc_layer(xq, ho, wo, w4a, b, act, out_dtype, out_tr=False,
               quad_out=False):
    n, _, r, c = xq.shape
    cout = w4a.shape[-1]
    if out_tr:
        out_spec = pl.BlockSpec((GB, cout, ho * wo), lambda i: (i, 0, 0))
        out_shape = jax.ShapeDtypeStruct((n, cout, ho * wo), out_dtype)
    elif quad_out:
        r2 = (ho // 2 + 2) * (wo // 2 + 1)
        out_spec = pl.BlockSpec((GB, 4, r2, cout), lambda i: (i, 0, 0, 0))
        out_shape = jax.ShapeDtypeStruct((n, 4, r2, cout), out_dtype)
    else:
        out_spec = pl.BlockSpec((GB, ho, wo, cout), lambda i: (i, 0, 0, 0))
        out_shape = jax.ShapeDtypeStruct((n, ho, wo, cout), out_dtype)
    return pl.pallas_call(
        functools.partial(_enc_body, ho=ho, wo=wo, act=act, out_tr=out_tr,
                          quad_out=quad_out),
        grid=(n // GB,),
        in_specs=[
            pl.BlockSpec((GB, 4, r, c), lambda i: (i, 0, 0, 0)),
            pl.BlockSpec((16 * c, cout), lambda i: (0, 0)),
            pl.BlockSpec((1, cout), lambda i: (0, 0)),
        ],
        out_specs=out_spec,
        out_shape=out_shape,
    )(xq, w4a, b.reshape(1, cout))


def _dec_w(dw):
    cin, cout = dw.shape[0], dw.shape[1]
    wt = jnp.flip(dw, (2, 3)).transpose(1, 0, 2, 3)
    wp = wt.transpose(2, 3, 1, 0)
    w9 = jnp.zeros((9, cin, 2, 2, cout), dw.dtype)
    for pr in (0, 1):
        for di in ((0, 1) if pr == 0 else (1, 2)):
            kh = 2 * di - pr
            for pc in (0, 1):
                for dj in ((0, 1) if pc == 0 else (1, 2)):
                    kw = 2 * dj - pc
                    w9 = w9.at[di * 3 + dj, :, pr, pc, :].set(wp[kh, kw])
    return w9.reshape(9, cin, 4 * cout).astype(BF)


def _vq_body(f32_ref, fbf_ref, embt_ref, emb_ref, idx_ref):
    f = f32_ref[0]
    m = jnp.dot(fbf_ref[0], embt_ref[...], preferred_element_type=F32)
    f2 = jnp.sum(f * f, axis=1, keepdims=True)
    e = emb_ref[...]
    e2 = jnp.sum(e * e, axis=1)[None, :]
    d = f2 + e2 - 2.0 * m
    dmin = jnp.min(d, axis=1, keepdims=True)
    col = lax.broadcasted_iota(jnp.int32, d.shape, 1)
    idx = jnp.min(jnp.where(d == dmin, col, K_CODES), axis=1)
    idx_ref[0] = idx.reshape(1, VQ_BLK)


def _vq_argmin():
    return pl.pallas_call(
        _vq_body,
        grid=(8,),
        in_specs=[
            pl.BlockSpec((1, VQ_BLK, D_CODE), lambda i: (i, 0, 0)),
            pl.BlockSpec((1, VQ_BLK, D_CODE), lambda i: (i, 0, 0)),
            pl.BlockSpec((D_CODE, K_CODES), lambda i: (0, 0)),
            pl.BlockSpec((K_CODES, D_CODE), lambda i: (0, 0)),
        ],
        out_specs=pl.BlockSpec((1, 1, VQ_BLK), lambda i: (i, 0, 0)),
        out_shape=jax.ShapeDtypeStruct((8, 1, VQ_BLK), jnp.int32),
    )


def _sc_gather_body(table_hbm, idx_hbm, out_hbm, idx_v, rows_v, sem, *, nc, bpw):
    wid = lax.axis_index("s") * nc + lax.axis_index("c")
    base = wid * bpw
    pltpu.sync_copy(idx_hbm.at[pl.ds(base, bpw)], idx_v)
    pltpu.async_copy(table_hbm.at[idx_v], rows_v, sem).wait()
    pltpu.sync_copy(rows_v, out_hbm.at[pl.ds(base, bpw)])


def _sc_gather(emb_f32, idx_pad):
    info = plsc.get_sparse_core_info()
    nc, ns = info.num_cores, info.num_subcores
    bpw = B_PAD // (nc * ns)
    k = pl.kernel(
        functools.partial(_sc_gather_body, nc=nc, bpw=bpw),
        mesh=plsc.VectorSubcoreMesh(core_axis_name="c", subcore_axis_name="s"),
        out_type=jax.ShapeDtypeStruct((B_PAD, D_CODE), F32),
        scratch_types=[
            pltpu.VMEM((bpw,), jnp.int32),
            pltpu.VMEM((bpw, D_CODE), F32),
            pltpu.SemaphoreType.DMA,
        ],
    )
    return k(emb_f32, idx_pad)


def _loss_body(q_ref, z_ref, part_ref, qn_ref):
    for g in range(GB):
        q = q_ref[g]
        dlt = q - z_ref[g]
        part_ref[g] = ((1.25 / (N_TOK * D_CODE))
                       * jnp.sum(dlt * dlt)).reshape(1, 1)
        qn_ref[g] = q.T.astype(BF)


def _loss_and_qt(qc, zc):
    return pl.pallas_call(
        _loss_body,
        grid=(N_IMG // GB,),
        in_specs=[
            pl.BlockSpec((GB, 256, 196), lambda i: (i, 0, 0)),
            pl.BlockSpec((GB, 256, 196), lambda i: (i, 0, 0)),
        ],
        out_specs=[
            pl.BlockSpec((GB, 1, 1), lambda i: (i, 0, 0)),
            pl.BlockSpec((GB, 196, 256), lambda i: (i, 0, 0)),
        ],
        out_shape=[
            jax.ShapeDtypeStruct((N_IMG, 1, 1), F32),
            jax.ShapeDtypeStruct((N_IMG, 196, 256), BF),
        ],
    )(qc, zc)


def kernel(x, w1, b1, w2, b2, w3, b3, w4, b4, emb,
           dw1, db1, dw2, db2, dw3, db3, dw4, db4):
    nhwc = lambda a: a.transpose(0, 2, 3, 1)

    h = _mm_layer(_enc_im2col(nhwc(x).astype(BF)), _enc_w(w1), b1, 'relu', BF,
                  quad_hw=(112, 112))
    h = _enc_layer(h, 56, 56, _enc_w(w2), b2, 'relu', BF, quad_out=True)
    h = _enc_layer(h, 28, 28, _enc_w(w3), b3, 'relu', BF, quad_out=True)
    zc = _enc_layer(h, 14, 14, _enc_w(w4), b4, 'none', F32,
                    out_tr=True)

    zf = zc.reshape(N_TOK, D_CODE)
    embt = emb.T.astype(BF)
    idx8 = _vq_argmin()(zf.reshape(8, VQ_BLK, D_CODE),
                        zf.astype(BF).reshape(8, VQ_BLK, D_CODE), embt, emb)
    indices = idx8.reshape(N_TOK, 1)
    idx_pad = jnp.concatenate(
        [idx8.reshape(N_TOK), jnp.zeros((B_PAD - N_TOK,), jnp.int32)])
    qf = _sc_gather(emb, idx_pad)[:N_TOK]
    parts, q_nhwc = _loss_and_qt(qf.reshape(N_IMG, 256, 196), zc)
    vq_loss = jnp.sum(parts)

    q_nhwc = q_nhwc.reshape(N_IMG, 14, 14, 256)
    h = _dec_layer(q_nhwc, _dec_w(dw1), jnp.tile(db1, 4), 'relu', BF, inter=True)
    h = _dec_layer(h, _dec_w(dw2), jnp.tile(db2, 4), 'relu', BF, inter=True)
    h = _dec_layer(h, _dec_w(dw3), jnp.tile(db3, 4), 'relu', BF, inter=True)
    y = _dec_layer(h, _dec_w(dw4), jnp.tile(db4, 4), 'sigmoid', F32,
                   inter='img')
    x_recon = y.reshape(N_IMG, 1, 224, 224)

    return (x_recon, vq_loss, indices)
